# baseline (speedup 1.0000x reference)
import functools

import jax
import jax.numpy as jnp
from jax import lax
from jax.experimental import pallas as pl
from jax.experimental.pallas import tpu as pltpu
from jax.experimental.pallas import tpu_sc as plsc

D = 128
HALF = 64
N_NODES = 10000
N_SPLITS = 2
_TILE = 16000
_ZBLK = 80
_OBLK = 200
_RS = 128


def _msg_body(x_ref, w2_ref, o_ref):
    xb = x_ref[...]
    x0 = xb[:, :HALF]
    x1 = xb[:, HALF:]
    w2 = w2_ref[...]
    dn = (((1,), (0,)), ((), ()))
    a = jnp.clip(lax.dot_general(xb, w2, dn, preferred_element_type=jnp.float32),
                 0.0, 1.0)
    x0r = jnp.concatenate([x0, x0], axis=1)
    x1r = jnp.concatenate([x1, x1], axis=1)
    o_ref[...] = x0r * a[:, :D] + x1r * a[:, D:]


def _messages(x, W2, start_tile, n_tiles):
    return pl.pallas_call(
        _msg_body,
        grid=(n_tiles,),
        in_specs=[
            pl.BlockSpec((_TILE, D), lambda i: (i + start_tile, 0)),
            pl.BlockSpec((D, 2 * D), lambda i: (0, 0)),
        ],
        out_specs=pl.BlockSpec((_TILE, D), lambda i: (i, 0)),
        out_shape=jax.ShapeDtypeStruct((n_tiles * _TILE, D), jnp.float32),
    )(x, W2)


def _make_sc_scatter(e_chunk, n_nodes):
    info = plsc.get_sparse_core_info()
    nc, ns = info.num_cores, info.num_subcores
    nw = nc * ns
    per_worker = e_chunk // nw
    assert per_worker * nw == e_chunk
    iters = per_worker // _RS
    tail = per_worker - iters * _RS
    assert iters >= 2 and tail % 8 == 0 and 0 < tail <= 128
    pairs = iters // 2
    odd = iters % 2
    assert n_nodes % _ZBLK == 0 and n_nodes % _OBLK == 0
    zblocks = n_nodes // _ZBLK
    zrounds = (zblocks + ns - 1) // ns
    oblocks = n_nodes // _OBLK
    orounds = (oblocks + ns - 1) // ns

    mesh = plsc.VectorSubcoreMesh(core_axis_name="c", subcore_axis_name="s")

    @functools.partial(
        pl.kernel,
        mesh=mesh,
        out_type=jax.ShapeDtypeStruct((nc, n_nodes, D), jnp.float32),
        scratch_types=[
            pltpu.VMEM_SHARED((n_nodes, D), jnp.float32),
            pltpu.VMEM((_RS, D), jnp.float32),
            pltpu.VMEM((_RS, D), jnp.float32),
            pltpu.VMEM((iters, _RS), jnp.int32),
            pltpu.VMEM((tail,), jnp.int32),
            pltpu.VMEM((_ZBLK, D), jnp.float32),
            pltpu.SemaphoreType.DMA,
            pltpu.SemaphoreType.DMA,
        ],
    )
    def sc_scatter(msgs_hbm, idx_hbm, tidx_hbm, out_hbm, acc, rows0, rows1,
                   idx_v, tidx_v, zbuf, ld0, ld1):
        c = lax.axis_index("c")
        s = lax.axis_index("s")
        zvec = jnp.zeros((16,), jnp.float32)

        def zero_row(r, carry):
            for j in range(D // 16):
                zbuf[r, pl.ds(j * 16, 16)] = zvec
            return carry

        lax.fori_loop(0, _ZBLK, zero_row, 0)
        for k in range(zrounds):
            blk = k * ns + s
            @pl.when(blk < zblocks)
            def _():
                pltpu.sync_copy(zbuf, acc.at[pl.ds(blk * _ZBLK, _ZBLK)])

        w = c * ns + s
        base0 = w * per_worker
        pltpu.sync_copy(idx_hbm.at[w], idx_v)
        pltpu.sync_copy(tidx_hbm.at[w], tidx_v)
        plsc.subcore_barrier()

        rows = (rows0, rows1)
        lds = (ld0, ld1)
        tb = 1 if odd else 0
        tail_ref = rows[tb].at[pl.ds(0, tail)]

        def load_start(chunk, b):
            pltpu.async_copy(
                msgs_hbm.at[pl.ds(base0 + chunk * _RS, _RS)], rows[b], lds[b])

        def load_wait(chunk, b):
            pltpu.make_async_copy(
                msgs_hbm.at[pl.ds(base0 + chunk * _RS, _RS)], rows[b], lds[b]
            ).wait()

        def tail_load_start():
            pltpu.async_copy(
                msgs_hbm.at[pl.ds(base0 + iters * _RS, tail)], tail_ref, lds[tb])

        def tail_load_wait():
            pltpu.make_async_copy(
                msgs_hbm.at[pl.ds(base0 + iters * _RS, tail)], tail_ref, lds[tb]
            ).wait()

        load_start(0, 0)

        def pair(p, carry):
            i = p * 2
            load_wait(i, 0)
            load_start(i + 1, 1)
            pltpu.sync_copy(rows[0], acc.at[idx_v.at[i]], add=True)
            load_wait(i + 1, 1)
            @pl.when(i + 2 < iters)
            def _():
                load_start(i + 2, 0)
            if not odd:
                @pl.when(p + 1 == pairs)
                def _():
                    tail_load_start()
            pltpu.sync_copy(rows[1], acc.at[idx_v.at[i + 1]], add=True)
            return carry

        lax.fori_loop(0, pairs, pair, 0)
        if odd:
            load_wait(iters - 1, 0)
            tail_load_start()
            pltpu.sync_copy(rows[0], acc.at[idx_v.at[iters - 1]], add=True)
        tail_load_wait()
        pltpu.sync_copy(tail_ref, acc.at[tidx_v], add=True)

        plsc.subcore_barrier()
        for k in range(orounds):
            blk = k * ns + s
            @pl.when(blk < oblocks)
            def _():
                pltpu.sync_copy(
                    acc.at[pl.ds(blk * _OBLK, _OBLK)],
                    out_hbm.at[c].at[pl.ds(blk * _OBLK, _OBLK)],
                )

    return sc_scatter


def _combine_body(*refs):
    o_ref = refs[-1]
    acc = None
    for p_ref in refs[:-1]:
        t = p_ref[0] + p_ref[1]
        acc = t if acc is None else acc + t
    o_ref[...] = acc


def _combine(parts, tile=1000):
    n = parts[0].shape[1]
    assert n % tile == 0
    spec = pl.BlockSpec((2, tile, D), lambda i: (0, i, 0))
    return pl.pallas_call(
        _combine_body,
        grid=(n // tile,),
        in_specs=[spec] * len(parts),
        out_specs=pl.BlockSpec((tile, D), lambda i: (i, 0)),
        out_shape=jax.ShapeDtypeStruct((n, D), jnp.float32),
    )(*parts)


def kernel(x, index, dim, dim_size, W):
    E = x.shape[0]
    info = plsc.get_sparse_core_info()
    nw = info.num_cores * info.num_subcores

    e0 = E * 3 // 10
    e_list = [e0, E - 2 * e0, e0]

    seg = jnp.minimum(index.astype(jnp.int32) + dim, dim_size - 1).astype(jnp.int32)
    wt = W.T
    z = jnp.zeros((HALF, D), jnp.float32)
    W2 = jnp.block([[wt, z], [z, wt]])

    parts = []
    off = 0
    for e_k in e_list:
        assert e_k % _TILE == 0 and e_k % (nw * 8) == 0
        pw = e_k // nw
        iters = pw // _RS
        seg_k = seg[off:off + e_k].reshape(nw, pw)
        seg_main = seg_k[:, : iters * _RS].reshape(nw, iters, _RS)
        seg_tail = seg_k[:, iters * _RS:]
        msgs_k = _messages(x, W2, off // _TILE, e_k // _TILE)
        parts.append(_make_sc_scatter(e_k, N_NODES)(msgs_k, seg_main, seg_tail))
        off += e_k
    return _combine(parts)


# speedup vs baseline: 1.0215x; 1.0215x over previous
import functools

import jax
import jax.numpy as jnp
from jax import lax
from jax.experimental import pallas as pl
from jax.experimental.pallas import tpu as pltpu
from jax.experimental.pallas import tpu_sc as plsc

D = 128
HALF = 64
N_NODES = 10000
N_SPLITS = 2
_TILE = 16000
_ZBLK = 80
_OBLK = 200
_RS = 128


def _msg_body(x_ref, w2_ref, o_ref):
    xb = x_ref[...]
    x0 = xb[:, :HALF]
    x1 = xb[:, HALF:]
    w2 = w2_ref[...]
    dn = (((1,), (0,)), ((), ()))
    a = jnp.clip(lax.dot_general(xb, w2, dn, preferred_element_type=jnp.float32),
                 0.0, 1.0)
    x0r = jnp.concatenate([x0, x0], axis=1)
    x1r = jnp.concatenate([x1, x1], axis=1)
    o_ref[...] = x0r * a[:, :D] + x1r * a[:, D:]


def _messages(x, W2, start_tile, n_tiles):
    return pl.pallas_call(
        _msg_body,
        grid=(n_tiles,),
        in_specs=[
            pl.BlockSpec((_TILE, D), lambda i: (i + start_tile, 0)),
            pl.BlockSpec((D, 2 * D), lambda i: (0, 0)),
        ],
        out_specs=pl.BlockSpec((_TILE, D), lambda i: (i, 0)),
        out_shape=jax.ShapeDtypeStruct((n_tiles * _TILE, D), jnp.float32),
    )(x, W2)


def _make_sc_scatter(e_chunk, n_nodes):
    info = plsc.get_sparse_core_info()
    nc, ns = info.num_cores, info.num_subcores
    nw = nc * ns
    per_worker = e_chunk // nw
    assert per_worker * nw == e_chunk
    iters = per_worker // _RS
    tail = per_worker - iters * _RS
    assert iters >= 2 and tail % 8 == 0 and 0 < tail <= 128
    pairs = iters // 2
    odd = iters % 2
    assert n_nodes % _ZBLK == 0 and n_nodes % _OBLK == 0
    zblocks = n_nodes // _ZBLK
    zrounds = (zblocks + ns - 1) // ns
    oblocks = n_nodes // _OBLK
    orounds = (oblocks + ns - 1) // ns

    mesh = plsc.VectorSubcoreMesh(core_axis_name="c", subcore_axis_name="s")

    @functools.partial(
        pl.kernel,
        mesh=mesh,
        out_type=jax.ShapeDtypeStruct((nc, n_nodes, D), jnp.float32),
        scratch_types=[
            pltpu.VMEM_SHARED((n_nodes, D), jnp.float32),
            pltpu.VMEM((_RS, D), jnp.float32),
            pltpu.VMEM((_RS, D), jnp.float32),
            pltpu.VMEM((iters, _RS), jnp.int32),
            pltpu.VMEM((tail,), jnp.int32),
            pltpu.VMEM((_ZBLK, D), jnp.float32),
            pltpu.SemaphoreType.DMA,
            pltpu.SemaphoreType.DMA,
        ],
    )
    def sc_scatter(msgs_hbm, idx_hbm, tidx_hbm, out_hbm, acc, rows0, rows1,
                   idx_v, tidx_v, zbuf, ld0, ld1):
        c = lax.axis_index("c")
        s = lax.axis_index("s")
        zvec = jnp.zeros((16,), jnp.float32)

        def zero_row(r, carry):
            for j in range(D // 16):
                zbuf[r, pl.ds(j * 16, 16)] = zvec
            return carry

        lax.fori_loop(0, _ZBLK, zero_row, 0)
        for k in range(zrounds):
            blk = k * ns + s
            @pl.when(blk < zblocks)
            def _():
                pltpu.sync_copy(zbuf, acc.at[pl.ds(blk * _ZBLK, _ZBLK)])

        w = c * ns + s
        base0 = w * per_worker
        pltpu.sync_copy(idx_hbm.at[w], idx_v)
        pltpu.sync_copy(tidx_hbm.at[w], tidx_v)
        plsc.subcore_barrier()

        rows = (rows0, rows1)
        lds = (ld0, ld1)
        tb = 1 if odd else 0
        tail_ref = rows[tb].at[pl.ds(0, tail)]

        def load_start(chunk, b):
            pltpu.async_copy(
                msgs_hbm.at[pl.ds(base0 + chunk * _RS, _RS)], rows[b], lds[b])

        def load_wait(chunk, b):
            pltpu.make_async_copy(
                msgs_hbm.at[pl.ds(base0 + chunk * _RS, _RS)], rows[b], lds[b]
            ).wait()

        def tail_load_start():
            pltpu.async_copy(
                msgs_hbm.at[pl.ds(base0 + iters * _RS, tail)], tail_ref, lds[tb])

        def tail_load_wait():
            pltpu.make_async_copy(
                msgs_hbm.at[pl.ds(base0 + iters * _RS, tail)], tail_ref, lds[tb]
            ).wait()

        load_start(0, 0)

        def pair(p, carry):
            i = p * 2
            load_wait(i, 0)
            load_start(i + 1, 1)
            pltpu.sync_copy(rows[0], acc.at[idx_v.at[i]], add=True)
            load_wait(i + 1, 1)
            @pl.when(i + 2 < iters)
            def _():
                load_start(i + 2, 0)
            if not odd:
                @pl.when(p + 1 == pairs)
                def _():
                    tail_load_start()
            pltpu.sync_copy(rows[1], acc.at[idx_v.at[i + 1]], add=True)
            return carry

        lax.fori_loop(0, pairs, pair, 0)
        if odd:
            load_wait(iters - 1, 0)
            tail_load_start()
            pltpu.sync_copy(rows[0], acc.at[idx_v.at[iters - 1]], add=True)
        tail_load_wait()
        pltpu.sync_copy(tail_ref, acc.at[tidx_v], add=True)

        plsc.subcore_barrier()
        for k in range(orounds):
            blk = k * ns + s
            @pl.when(blk < oblocks)
            def _():
                pltpu.sync_copy(
                    acc.at[pl.ds(blk * _OBLK, _OBLK)],
                    out_hbm.at[c].at[pl.ds(blk * _OBLK, _OBLK)],
                )

    return sc_scatter


def _combine_body(*refs):
    o_ref = refs[-1]
    acc = None
    for p_ref in refs[:-1]:
        t = p_ref[0] + p_ref[1]
        acc = t if acc is None else acc + t
    o_ref[...] = acc


def _combine(parts, tile=1000):
    n = parts[0].shape[1]
    assert n % tile == 0
    spec = pl.BlockSpec((2, tile, D), lambda i: (0, i, 0))
    return pl.pallas_call(
        _combine_body,
        grid=(n // tile,),
        in_specs=[spec] * len(parts),
        out_specs=pl.BlockSpec((tile, D), lambda i: (i, 0)),
        out_shape=jax.ShapeDtypeStruct((n, D), jnp.float32),
    )(*parts)


def kernel(x, index, dim, dim_size, W):
    E = x.shape[0]
    info = plsc.get_sparse_core_info()
    nw = info.num_cores * info.num_subcores

    e0 = E // 2
    e_list = [e0, E - e0]

    seg = jnp.minimum(index.astype(jnp.int32) + dim, dim_size - 1).astype(jnp.int32)
    wt = W.T
    z = jnp.zeros((HALF, D), jnp.float32)
    W2 = jnp.block([[wt, z], [z, wt]])

    parts = []
    off = 0
    for e_k in e_list:
        assert e_k % _TILE == 0 and e_k % (nw * 8) == 0
        pw = e_k // nw
        iters = pw // _RS
        seg_k = seg[off:off + e_k].reshape(nw, pw)
        seg_main = seg_k[:, : iters * _RS].reshape(nw, iters, _RS)
        seg_tail = seg_k[:, iters * _RS:]
        msgs_k = _messages(x, W2, off // _TILE, e_k // _TILE)
        parts.append(_make_sc_scatter(e_k, N_NODES)(msgs_k, seg_main, seg_tail))
        off += e_k
    return _combine(parts)
